# baseline (speedup 1.0000x reference)
import jax
import jax.numpy as jnp
import numpy as np
from jax.experimental import pallas as pl
from jax.experimental.pallas import tpu as pltpu

_SUBL = 8
_ROWB = 256
_CH = 64
_BIG = 1.0e30


def _const_mask(fn, n_proj):
    rows = jax.lax.broadcasted_iota(jnp.int32, (_SUBL, n_proj), 0)
    return fn(rows)


def _subvreg_step(vs, k, j, n_proj, asc_static=None):
    lowm = _const_mask(lambda r: (r & j) == 0, n_proj)
    out = []
    for idx, v in enumerate(vs):
        if j == _SUBL // 2:
            p = pltpu.roll(v, j, 0)
        else:
            pm = pltpu.roll(v, _SUBL - j, 0)
            pp = pltpu.roll(v, j, 0)
            p = jnp.where(lowm, pm, pp)
        mn = jnp.minimum(v, p)
        mx = jnp.maximum(v, p)
        if asc_static is None:
            tm = _const_mask(
                lambda r: (((idx * _SUBL + r) & k) == 0) == ((r & j) == 0),
                n_proj)
        elif asc_static:
            tm = lowm
        else:
            tm = jnp.logical_not(lowm)
        out.append(jnp.where(tm, mn, mx))
    return out


def _pair_step(vs, k, j, asc_static=None):
    j8 = j // _SUBL
    out = list(vs)
    for a in range(len(vs)):
        if a & j8:
            continue
        b = a + j8
        mn = jnp.minimum(vs[a], vs[b])
        mx = jnp.maximum(vs[a], vs[b])
        if asc_static is None:
            asc = ((a * _SUBL) & k) == 0
        else:
            asc = asc_static
        if asc:
            out[a], out[b] = mn, mx
        else:
            out[a], out[b] = mx, mn
    return out


def _merge_block(vs, n_proj, asc):
    j = _ROWB // 2
    while j >= 1:
        if j < _SUBL:
            vs = _subvreg_step(vs, 0, j, n_proj, asc_static=asc)
        else:
            vs = _pair_step(vs, 0, j, asc_static=asc)
        j //= 2
    return vs


def _make_kernel(hw, n_pad, c_chunk, c_chunks, n_proj):
    nv = _ROWB // _SUBL
    hw_chunks = hw // 1024 if hw % 1024 == 0 else 1
    hw_chunk = hw // hw_chunks
    dn = (((0,), (0,)), ((), ()))
    special = (hw == 9216 and n_pad == 16384 and _ROWB == 256)

    def load_block(ref, b0):
        return [ref[pl.ds(b0 + i * _SUBL, _SUBL), :] for i in range(nv)]

    def store_block(ref, b0, vs):
        for i, v in enumerate(vs):
            ref[pl.ds(b0 + i * _SUBL, _SUBL), :] = v

    def kern(mask_ref, p1_ref, p2_ref, proj_ref, out_ref, s1_ref, s2_ref):
        ci = pl.program_id(1)

        def mm_body(t, _):
            r0 = t * hw_chunk
            for src, dst in ((p1_ref, s1_ref), (p2_ref, s2_ref)):
                lhs = src[0, :, pl.ds(r0, hw_chunk)]
                pc = proj_ref[pl.ds(ci * c_chunk, c_chunk), :]
                v = jax.lax.dot_general(
                    lhs, pc, dn, preferred_element_type=jnp.float32)

                @pl.when(ci == 0)
                def _():
                    dst[pl.ds(r0, hw_chunk), :] = v

                @pl.when(ci > 0)
                def _():
                    dst[pl.ds(r0, hw_chunk), :] += v
            return 0

        jax.lax.fori_loop(0, hw_chunks, mm_body, 0)

        @pl.when(ci == 0)
        def _init_pad():
            if n_pad > hw:
                pad = jnp.full((n_pad - hw, n_proj), jnp.inf, jnp.float32)
                s1_ref[hw:, :] = pad
                s2_ref[hw:, :] = pad

        @pl.when(ci == c_chunks - 1)
        def _finish():
            maskf = mask_ref[0]
            nf = jnp.sum(maskf)
            pm = proj_ref[...]
            scale = 1.0 / jnp.maximum(
                jnp.sqrt(jnp.sum(pm * pm, axis=0, keepdims=True)), 1e-12)
            ones = jnp.ones((1, n_proj), jnp.float32)

            n_blocks = n_pad // _ROWB
            data_blocks = -(-hw // _ROWB)

            if not special:
                def scale_body(t, _):
                    r0 = t * hw_chunk
                    mrow = mask_ref[0, :, pl.ds(r0, hw_chunk)]
                    bslice = jax.lax.dot_general(
                        (1.0 - mrow) * _BIG, ones, dn,
                        preferred_element_type=jnp.float32)
                    s1_ref[pl.ds(r0, hw_chunk), :] = (
                        s1_ref[pl.ds(r0, hw_chunk), :] * scale + bslice)
                    s2_ref[pl.ds(r0, hw_chunk), :] = (
                        s2_ref[pl.ds(r0, hw_chunk), :] * scale + bslice)
                    return 0

                jax.lax.fori_loop(0, hw_chunks, scale_body, 0)

            def local_a(ref):
                def body(bi, _):
                    b0 = pl.multiple_of(bi * _ROWB, _ROWB)
                    vs = load_block(ref, b0)
                    if special:
                        mrow = mask_ref[0, :, pl.ds(b0, _ROWB)]
                        bigm = jax.lax.dot_general(
                            (1.0 - mrow) * _BIG, ones, dn,
                            preferred_element_type=jnp.float32)
                        vs = [v * scale + bigm[i * _SUBL:(i + 1) * _SUBL, :]
                              for i, v in enumerate(vs)]
                    k = 2
                    while k <= _ROWB // 2:
                        j = k // 2
                        while j >= 1:
                            if j < _SUBL:
                                vs = _subvreg_step(vs, k, j, n_proj)
                            else:
                                vs = _pair_step(vs, k, j)
                            j //= 2
                        k *= 2
                    asc = (b0 & _ROWB) == 0
                    vs = jax.lax.cond(
                        asc,
                        lambda *a: tuple(_merge_block(list(a), n_proj, True)),
                        lambda *a: tuple(_merge_block(list(a), n_proj, False)),
                        *vs)
                    store_block(ref, b0, list(vs))
                    return 0
                nb = data_blocks if special else n_blocks
                jax.lax.fori_loop(0, nb, body, 0)

            def local_b(ref, kk, nb, remap):
                def body(bi, _):
                    if remap:
                        bi = jnp.where(
                            (kk == 8192) & (bi >= data_blocks),
                            bi + (n_blocks - data_blocks - 4), bi)
                    b0 = pl.multiple_of(bi * _ROWB, _ROWB)
                    asc = (b0 & kk) == 0
                    vs = load_block(ref, b0)
                    vs = jax.lax.cond(
                        asc,
                        lambda *a: tuple(_merge_block(list(a), n_proj, True)),
                        lambda *a: tuple(_merge_block(list(a), n_proj, False)),
                        *vs)
                    store_block(ref, b0, list(vs))
                    return 0
                jax.lax.fori_loop(0, nb, body, 0)

            def global_pass(kk, jj, limit):
                jc = jj // _CH

                def body(t, _):
                    lo = jnp.bitwise_and(t, jc - 1)
                    ra = ((t - lo) * 2 + lo) * _CH
                    rb = ra + jj

                    @pl.when(rb < limit)
                    def _():
                        asc = (ra & kk) == 0
                        amn = jnp.where(asc, ra, rb)
                        amx = jnp.where(asc, rb, ra)
                        for ref in (s1_ref, s2_ref):
                            va = ref[pl.ds(ra, _CH), :]
                            vb = ref[pl.ds(rb, _CH), :]
                            mn = jnp.minimum(va, vb)
                            mx = jnp.maximum(va, vb)
                            ref[pl.ds(amn, _CH), :] = mn
                            ref[pl.ds(amx, _CH), :] = mx
                    return 0

                jax.lax.fori_loop(0, n_pad // (2 * _CH), body, 0)

            local_a(s1_ref)
            local_a(s2_ref)

            if n_pad > _ROWB:
                n_levels = (n_pad // _ROWB).bit_length() - 1

                def level_body(t, _):
                    kk = jnp.int32(2 * _ROWB) << t

                    def pass_body(s, _):
                        jj = (kk // 2) >> s
                        if special:
                            limit = jnp.where(kk <= 4096, hw, n_pad)
                        else:
                            limit = jnp.int32(n_pad)
                        global_pass(kk, jj, limit)
                        return 0

                    jax.lax.fori_loop(0, t + 1, pass_body, 0)

                    if special:
                        nb = jnp.where(kk <= 4096, data_blocks,
                                       jnp.where(kk == 8192,
                                                 data_blocks + 4, n_blocks))
                        local_b(s1_ref, kk, nb, True)
                        local_b(s2_ref, kk, nb, True)
                    else:
                        local_b(s1_ref, kk, n_blocks, False)
                        local_b(s2_ref, kk, n_blocks, False)
                    return 0

                jax.lax.fori_loop(0, n_levels, level_body, 0)

            n_i = nf.astype(jnp.int32)
            riota = jax.lax.broadcasted_iota(jnp.int32, (_CH, n_proj), 0)

            def red_body(t, acc):
                r0 = t * _CH
                x1 = s1_ref[pl.ds(r0, _CH), :]
                x2 = s2_ref[pl.ds(r0, _CH), :]
                valid = (riota + r0) < n_i
                d = jnp.where(valid, jnp.abs(x1 - x2), 0.0)
                return acc + jnp.sum(d)

            total = jax.lax.fori_loop(0, hw // _CH, red_body,
                                      jnp.float32(0.0))
            denom = jnp.maximum(nf * n_proj, 1.0)
            contrib = jnp.where(nf > 0, total / denom, 0.0)
            out_ref[0] = jnp.full((8, 128), contrib, jnp.float32)

    return kern


def kernel(content_features1, content_features2, gt_change_map, projections):
    b, c, h, w = content_features1.shape
    hw = h * w
    n_proj = projections.shape[1]
    ho, wo = gt_change_map.shape[2], gt_change_map.shape[3]

    r = (jnp.arange(h) * ho) // h
    cl = (jnp.arange(w) * wo) // w
    gt_small = gt_change_map[:, 0][:, r[:, None], cl[None, :]]
    maskf = (gt_small == 0).astype(jnp.float32).reshape(b, 1, hw)

    p1 = content_features1.reshape(b, c, hw)
    p2 = content_features2.reshape(b, c, hw)

    n_pad = 1 << (hw - 1).bit_length()
    n_pad = max(n_pad, _ROWB)
    c_chunks = 2 if c % 2 == 0 else (3 if c % 3 == 0 else 1)
    c_chunk = c // c_chunks

    kern = _make_kernel(hw, n_pad, c_chunk, c_chunks, n_proj)

    out = pl.pallas_call(
        kern,
        grid=(b, c_chunks),
        in_specs=[
            pl.BlockSpec((1, 1, hw), lambda i, ci: (i, 0, 0)),
            pl.BlockSpec((1, c_chunk, hw), lambda i, ci: (i, ci, 0)),
            pl.BlockSpec((1, c_chunk, hw), lambda i, ci: (i, ci, 0)),
            pl.BlockSpec((c, n_proj), lambda i, ci: (0, 0)),
        ],
        out_specs=pl.BlockSpec((1, 8, 128), lambda i, ci: (i, 0, 0)),
        out_shape=jax.ShapeDtypeStruct((b, 8, 128), jnp.float32),
        scratch_shapes=[
            pltpu.VMEM((n_pad, n_proj), jnp.float32),
            pltpu.VMEM((n_pad, n_proj), jnp.float32),
        ],
        compiler_params=pltpu.CompilerParams(
            dimension_semantics=("parallel", "arbitrary"),
        ),
    )(maskf, p1, p2, projections)

    return jnp.sum(out[:, 0, 0]) / b


# speedup vs baseline: 1.1465x; 1.1465x over previous
import jax
import jax.numpy as jnp
import numpy as np
from jax.experimental import pallas as pl
from jax.experimental.pallas import tpu as pltpu

_SUBL = 8
_ROWB = 256
_CH = 64
_BIG = 1.0e30


def _const_mask(fn, n_proj):
    rows = jax.lax.broadcasted_iota(jnp.int32, (_SUBL, n_proj), 0)
    return fn(rows)


def _subvreg_step(vs, k, j, n_proj, asc_static=None):
    lowm = _const_mask(lambda r: (r & j) == 0, n_proj)
    out = []
    for idx, v in enumerate(vs):
        if j == _SUBL // 2:
            p = pltpu.roll(v, j, 0)
        else:
            pm = pltpu.roll(v, _SUBL - j, 0)
            pp = pltpu.roll(v, j, 0)
            p = jnp.where(lowm, pm, pp)
        mn = jnp.minimum(v, p)
        mx = jnp.maximum(v, p)
        if asc_static is None:
            tm = _const_mask(
                lambda r: (((idx * _SUBL + r) & k) == 0) == ((r & j) == 0),
                n_proj)
        elif asc_static:
            tm = lowm
        else:
            tm = jnp.logical_not(lowm)
        out.append(jnp.where(tm, mn, mx))
    return out


def _pair_step(vs, k, j, asc_static=None):
    j8 = j // _SUBL
    out = list(vs)
    for a in range(len(vs)):
        if a & j8:
            continue
        b = a + j8
        mn = jnp.minimum(vs[a], vs[b])
        mx = jnp.maximum(vs[a], vs[b])
        if asc_static is None:
            asc = ((a * _SUBL) & k) == 0
        else:
            asc = asc_static
        if asc:
            out[a], out[b] = mn, mx
        else:
            out[a], out[b] = mx, mn
    return out


def _merge_block(vs, n_proj, asc):
    j = _ROWB // 2
    while j >= 1:
        if j < _SUBL:
            vs = _subvreg_step(vs, 0, j, n_proj, asc_static=asc)
        else:
            vs = _pair_step(vs, 0, j, asc_static=asc)
        j //= 2
    return vs


def _make_kernel(hw, n_pad, c_chunk, c_chunks, n_proj):
    nv = _ROWB // _SUBL
    hw_chunks = hw // 1024 if hw % 1024 == 0 else 1
    hw_chunk = hw // hw_chunks
    dn = (((0,), (0,)), ((), ()))
    special = (hw == 9216 and n_pad == 16384 and _ROWB == 256)

    def load_block(ref, b0):
        return [ref[pl.ds(b0 + i * _SUBL, _SUBL), :] for i in range(nv)]

    def store_block(ref, b0, vs):
        for i, v in enumerate(vs):
            ref[pl.ds(b0 + i * _SUBL, _SUBL), :] = v

    def kern(mask_ref, p1_ref, p2_ref, proj_ref, out_ref, s1_ref, s2_ref):
        ci = pl.program_id(1)

        def mm_body(t, _):
            r0 = t * hw_chunk
            for src, dst in ((p1_ref, s1_ref), (p2_ref, s2_ref)):
                lhs = src[0, :, pl.ds(r0, hw_chunk)]
                pc = proj_ref[pl.ds(ci * c_chunk, c_chunk), :]
                v = jax.lax.dot_general(
                    lhs, pc, dn, preferred_element_type=jnp.float32)

                @pl.when(ci == 0)
                def _():
                    dst[pl.ds(r0, hw_chunk), :] = v

                @pl.when(ci > 0)
                def _():
                    dst[pl.ds(r0, hw_chunk), :] += v
            return 0

        jax.lax.fori_loop(0, hw_chunks, mm_body, 0)

        @pl.when(ci == 0)
        def _init_pad():
            if n_pad > hw and not special:
                pad = jnp.full((n_pad - hw, n_proj), jnp.inf, jnp.float32)
                s1_ref[hw:, :] = pad
                s2_ref[hw:, :] = pad

        @pl.when(ci == c_chunks - 1)
        def _finish():
            maskf = mask_ref[0]
            nf = jnp.sum(maskf)
            pm = proj_ref[...]
            scale = 1.0 / jnp.maximum(
                jnp.sqrt(jnp.sum(pm * pm, axis=0, keepdims=True)), 1e-12)
            ones = jnp.ones((1, n_proj), jnp.float32)

            n_blocks = n_pad // _ROWB
            data_blocks = -(-hw // _ROWB)

            if True:
                def scale_body(t, _):
                    r0 = t * hw_chunk
                    mrow = mask_ref[0, :, pl.ds(r0, hw_chunk)]
                    bslice = jax.lax.dot_general(
                        (1.0 - mrow) * _BIG, ones, dn,
                        preferred_element_type=jnp.float32)
                    if special:
                        sgn = jnp.where(t == hw_chunks - 1, -1.0, 1.0)
                    else:
                        sgn = 1.0
                    s1_ref[pl.ds(r0, hw_chunk), :] = (
                        s1_ref[pl.ds(r0, hw_chunk), :] * scale + bslice) * sgn
                    s2_ref[pl.ds(r0, hw_chunk), :] = (
                        s2_ref[pl.ds(r0, hw_chunk), :] * scale + bslice) * sgn
                    return 0

                jax.lax.fori_loop(0, hw_chunks, scale_body, 0)

            def local_a(ref):
                def body(bi, _):
                    b0 = pl.multiple_of(bi * _ROWB, _ROWB)
                    vs = load_block(ref, b0)
                    k = 2
                    while k <= _ROWB // 2:
                        j = k // 2
                        while j >= 1:
                            if j < _SUBL:
                                vs = _subvreg_step(vs, k, j, n_proj)
                            else:
                                vs = _pair_step(vs, k, j)
                            j //= 2
                        k *= 2
                    asc = (b0 & _ROWB) == 0
                    vs = jax.lax.cond(
                        asc,
                        lambda *a: tuple(_merge_block(list(a), n_proj, True)),
                        lambda *a: tuple(_merge_block(list(a), n_proj, False)),
                        *vs)
                    store_block(ref, b0, list(vs))
                    return 0
                nb = data_blocks if special else n_blocks
                jax.lax.fori_loop(0, nb, body, 0)

            def local_b(ref, kk, nb, remap):
                def body(bi, _):
                    if remap:
                        bi = jnp.where(
                            (kk == 8192) & (bi >= data_blocks),
                            bi + (n_blocks - data_blocks - 4), bi)
                    b0 = pl.multiple_of(bi * _ROWB, _ROWB)
                    asc = (b0 & kk) == 0
                    vs = load_block(ref, b0)
                    vs = jax.lax.cond(
                        asc,
                        lambda *a: tuple(_merge_block(list(a), n_proj, True)),
                        lambda *a: tuple(_merge_block(list(a), n_proj, False)),
                        *vs)
                    store_block(ref, b0, list(vs))
                    return 0
                jax.lax.fori_loop(0, nb, body, 0)

            def global_pass(kk, jj, limit):
                jc = jj // _CH

                def body(t, _):
                    lo = jnp.bitwise_and(t, jc - 1)
                    ra = ((t - lo) * 2 + lo) * _CH
                    rb = ra + jj

                    @pl.when(rb < limit)
                    def _():
                        asc = (ra & kk) == 0
                        amn = jnp.where(asc, ra, rb)
                        amx = jnp.where(asc, rb, ra)
                        for ref in (s1_ref, s2_ref):
                            va = ref[pl.ds(ra, _CH), :]
                            vb = ref[pl.ds(rb, _CH), :]
                            mn = jnp.minimum(va, vb)
                            mx = jnp.maximum(va, vb)
                            ref[pl.ds(amn, _CH), :] = mn
                            ref[pl.ds(amx, _CH), :] = mx
                    return 0

                jax.lax.fori_loop(0, n_pad // (2 * _CH), body, 0)

            local_a(s1_ref)
            local_a(s2_ref)

            if special:
                def level_body(t, _):
                    kk = jnp.int32(2 * _ROWB) << t

                    def pass_body(s, _):
                        jj = (kk // 2) >> s
                        global_pass(kk, jj,
                                    jnp.where(kk <= 1024, hw, 8192))
                        return 0

                    jax.lax.fori_loop(0, t + 1, pass_body, 0)
                    nb = jnp.where(kk <= 1024, data_blocks, data_blocks - 4)
                    local_b(s1_ref, kk, nb, False)
                    local_b(s2_ref, kk, nb, False)
                    return 0

                jax.lax.fori_loop(0, 5, level_body, 0)

                def shift_body(cc, _):
                    ra = 8192 - 1024 + cc * _CH
                    rb = 8192 + cc * _CH
                    for ref in (s1_ref, s2_ref):
                        va = ref[pl.ds(ra, _CH), :]
                        vb = -ref[pl.ds(rb, _CH), :]
                        ref[pl.ds(ra, _CH), :] = jnp.minimum(va, vb)
                        ref[pl.ds(rb, _CH), :] = jnp.maximum(va, vb)
                    return 0

                jax.lax.fori_loop(0, 1024 // _CH, shift_body, 0)

                def fin_pass(s, _):
                    global_pass(jnp.int32(n_pad), jnp.int32(4096) >> s,
                                jnp.int32(hw))
                    return 0

                jax.lax.fori_loop(0, 5, fin_pass, 0)

                def fin_local(ref):
                    def body(bi, _):
                        b0 = pl.multiple_of(bi * _ROWB, _ROWB)
                        vs = load_block(ref, b0)
                        vs = _merge_block(vs, n_proj, True)
                        store_block(ref, b0, vs)
                        return 0
                    jax.lax.fori_loop(0, data_blocks, body, 0)

                fin_local(s1_ref)
                fin_local(s2_ref)
            elif n_pad > _ROWB:
                n_levels = (n_pad // _ROWB).bit_length() - 1

                def level_body(t, _):
                    kk = jnp.int32(2 * _ROWB) << t

                    def pass_body(s, _):
                        jj = (kk // 2) >> s
                        global_pass(kk, jj, jnp.int32(n_pad))
                        return 0

                    jax.lax.fori_loop(0, t + 1, pass_body, 0)
                    local_b(s1_ref, kk, n_blocks, False)
                    local_b(s2_ref, kk, n_blocks, False)
                    return 0

                jax.lax.fori_loop(0, n_levels, level_body, 0)

            n_i = nf.astype(jnp.int32)
            riota = jax.lax.broadcasted_iota(jnp.int32, (_CH, n_proj), 0)

            def red_body(t, acc):
                r0 = t * _CH
                x1 = s1_ref[pl.ds(r0, _CH), :]
                x2 = s2_ref[pl.ds(r0, _CH), :]
                valid = (riota + r0) < n_i
                d = jnp.where(valid, jnp.abs(x1 - x2), 0.0)
                return acc + jnp.sum(d)

            total = jax.lax.fori_loop(0, hw // _CH, red_body,
                                      jnp.float32(0.0))
            denom = jnp.maximum(nf * n_proj, 1.0)
            contrib = jnp.where(nf > 0, total / denom, 0.0)
            out_ref[0] = jnp.full((8, 128), contrib, jnp.float32)

    return kern


def kernel(content_features1, content_features2, gt_change_map, projections):
    b, c, h, w = content_features1.shape
    hw = h * w
    n_proj = projections.shape[1]
    ho, wo = gt_change_map.shape[2], gt_change_map.shape[3]

    r = (jnp.arange(h) * ho) // h
    cl = (jnp.arange(w) * wo) // w
    gt_small = gt_change_map[:, 0][:, r[:, None], cl[None, :]]
    maskf = (gt_small == 0).astype(jnp.float32).reshape(b, 1, hw)

    p1 = content_features1.reshape(b, c, hw)
    p2 = content_features2.reshape(b, c, hw)

    n_pad = 1 << (hw - 1).bit_length()
    n_pad = max(n_pad, _ROWB)
    c_chunks = 2 if c % 2 == 0 else (3 if c % 3 == 0 else 1)
    c_chunk = c // c_chunks

    kern = _make_kernel(hw, n_pad, c_chunk, c_chunks, n_proj)

    out = pl.pallas_call(
        kern,
        grid=(b, c_chunks),
        in_specs=[
            pl.BlockSpec((1, 1, hw), lambda i, ci: (i, 0, 0)),
            pl.BlockSpec((1, c_chunk, hw), lambda i, ci: (i, ci, 0)),
            pl.BlockSpec((1, c_chunk, hw), lambda i, ci: (i, ci, 0)),
            pl.BlockSpec((c, n_proj), lambda i, ci: (0, 0)),
        ],
        out_specs=pl.BlockSpec((1, 8, 128), lambda i, ci: (i, 0, 0)),
        out_shape=jax.ShapeDtypeStruct((b, 8, 128), jnp.float32),
        scratch_shapes=[
            pltpu.VMEM((n_pad, n_proj), jnp.float32),
            pltpu.VMEM((n_pad, n_proj), jnp.float32),
        ],
        compiler_params=pltpu.CompilerParams(
            dimension_semantics=("parallel", "arbitrary"),
        ),
    )(maskf, p1, p2, projections)

    return jnp.sum(out[:, 0, 0]) / b


# speedup vs baseline: 1.2439x; 1.0849x over previous
import jax
import jax.numpy as jnp
import numpy as np
from jax.experimental import pallas as pl
from jax.experimental.pallas import tpu as pltpu

_SUBL = 8
_ROWB = 256
_CH = 64
_BIG = 1.0e30


def _const_mask(fn, n_proj):
    rows = jax.lax.broadcasted_iota(jnp.int32, (_SUBL, n_proj), 0)
    return fn(rows)


def _subvreg_step(vs, k, j, n_proj, asc_static=None):
    lowm = _const_mask(lambda r: (r & j) == 0, n_proj)
    out = []
    for idx, v in enumerate(vs):
        if j == _SUBL // 2:
            p = pltpu.roll(v, j, 0)
        else:
            pm = pltpu.roll(v, _SUBL - j, 0)
            pp = pltpu.roll(v, j, 0)
            p = jnp.where(lowm, pm, pp)
        mn = jnp.minimum(v, p)
        mx = jnp.maximum(v, p)
        if asc_static is None:
            tm = _const_mask(
                lambda r: (((idx * _SUBL + r) & k) == 0) == ((r & j) == 0),
                n_proj)
        elif asc_static:
            tm = lowm
        else:
            tm = jnp.logical_not(lowm)
        out.append(jnp.where(tm, mn, mx))
    return out


def _pair_step(vs, k, j, asc_static=None):
    j8 = j // _SUBL
    out = list(vs)
    for a in range(len(vs)):
        if a & j8:
            continue
        b = a + j8
        mn = jnp.minimum(vs[a], vs[b])
        mx = jnp.maximum(vs[a], vs[b])
        if asc_static is None:
            asc = ((a * _SUBL) & k) == 0
        else:
            asc = asc_static
        if asc:
            out[a], out[b] = mn, mx
        else:
            out[a], out[b] = mx, mn
    return out


def _merge_block(vs, n_proj, asc):
    j = _ROWB // 2
    while j >= 1:
        if j < _SUBL:
            vs = _subvreg_step(vs, 0, j, n_proj, asc_static=asc)
        else:
            vs = _pair_step(vs, 0, j, asc_static=asc)
        j //= 2
    return vs


def _make_kernel(hw, n_pad, c_chunk, c_chunks, n_proj):
    nv = _ROWB // _SUBL
    hw_chunks = hw // 1024 if hw % 1024 == 0 else 1
    hw_chunk = hw // hw_chunks
    dn = (((0,), (0,)), ((), ()))
    special = (hw == 9216 and n_pad == 16384 and _ROWB == 256)

    def load_block(ref, b0):
        return [ref[pl.ds(b0 + i * _SUBL, _SUBL), :] for i in range(nv)]

    def store_block(ref, b0, vs):
        for i, v in enumerate(vs):
            ref[pl.ds(b0 + i * _SUBL, _SUBL), :] = v

    def kern(mask_ref, p1_ref, p2_ref, proj_ref, out_ref, s1_ref, s2_ref):
        ci = pl.program_id(1)

        def mm_body(t, _):
            r0 = t * hw_chunk
            for src, dst in ((p1_ref, s1_ref), (p2_ref, s2_ref)):
                lhs = src[0, :, pl.ds(r0, hw_chunk)]
                pc = proj_ref[pl.ds(ci * c_chunk, c_chunk), :]
                v = jax.lax.dot_general(
                    lhs, pc, dn, preferred_element_type=jnp.float32)

                @pl.when(ci == 0)
                def _():
                    dst[pl.ds(r0, hw_chunk), :] = v

                @pl.when(ci > 0)
                def _():
                    dst[pl.ds(r0, hw_chunk), :] += v
            return 0

        jax.lax.fori_loop(0, hw_chunks, mm_body, 0)

        @pl.when(ci == 0)
        def _init_pad():
            if n_pad > hw and not special:
                pad = jnp.full((n_pad - hw, n_proj), jnp.inf, jnp.float32)
                s1_ref[hw:, :] = pad
                s2_ref[hw:, :] = pad

        @pl.when(ci == c_chunks - 1)
        def _finish():
            maskf = mask_ref[0]
            nf = jnp.sum(maskf)
            pm = proj_ref[...]
            scale = 1.0 / jnp.maximum(
                jnp.sqrt(jnp.sum(pm * pm, axis=0, keepdims=True)), 1e-12)
            ones = jnp.ones((1, n_proj), jnp.float32)

            n_blocks = n_pad // _ROWB
            data_blocks = -(-hw // _ROWB)

            if True:
                def scale_body(t, _):
                    r0 = t * hw_chunk
                    mrow = mask_ref[0, :, pl.ds(r0, hw_chunk)]
                    bslice = jax.lax.dot_general(
                        (1.0 - mrow) * _BIG, ones, dn,
                        preferred_element_type=jnp.float32)
                    if special:
                        sgn = jnp.where(t == hw_chunks - 1, -1.0, 1.0)
                    else:
                        sgn = 1.0
                    s1_ref[pl.ds(r0, hw_chunk), :] = (
                        s1_ref[pl.ds(r0, hw_chunk), :] * scale + bslice) * sgn
                    s2_ref[pl.ds(r0, hw_chunk), :] = (
                        s2_ref[pl.ds(r0, hw_chunk), :] * scale + bslice) * sgn
                    return 0

                jax.lax.fori_loop(0, hw_chunks, scale_body, 0)

            def local_a(ref):
                def body(bi, _):
                    b0 = pl.multiple_of(bi * _ROWB, _ROWB)
                    vs = load_block(ref, b0)
                    k = 2
                    while k <= _ROWB // 2:
                        j = k // 2
                        while j >= 1:
                            if j < _SUBL:
                                vs = _subvreg_step(vs, k, j, n_proj)
                            else:
                                vs = _pair_step(vs, k, j)
                            j //= 2
                        k *= 2
                    asc = (b0 & _ROWB) == 0
                    vs = jax.lax.cond(
                        asc,
                        lambda *a: tuple(_merge_block(list(a), n_proj, True)),
                        lambda *a: tuple(_merge_block(list(a), n_proj, False)),
                        *vs)
                    store_block(ref, b0, list(vs))
                    return 0
                nb = data_blocks if special else n_blocks
                jax.lax.fori_loop(0, nb, body, 0)

            def local_b(ref, kk, nb, remap):
                def body(bi, _):
                    if remap:
                        bi = jnp.where(
                            (kk == 8192) & (bi >= data_blocks),
                            bi + (n_blocks - data_blocks - 4), bi)
                    b0 = pl.multiple_of(bi * _ROWB, _ROWB)
                    asc = (b0 & kk) == 0
                    vs = load_block(ref, b0)
                    vs = jax.lax.cond(
                        asc,
                        lambda *a: tuple(_merge_block(list(a), n_proj, True)),
                        lambda *a: tuple(_merge_block(list(a), n_proj, False)),
                        *vs)
                    store_block(ref, b0, list(vs))
                    return 0
                jax.lax.fori_loop(0, nb, body, 0)

            def global_pass(kk, jj, limit):
                jc = jj // _CH

                def body(t, _):
                    lo = jnp.bitwise_and(t, jc - 1)
                    ra = ((t - lo) * 2 + lo) * _CH
                    rb = ra + jj

                    @pl.when(rb < limit)
                    def _():
                        asc = (ra & kk) == 0
                        amn = jnp.where(asc, ra, rb)
                        amx = jnp.where(asc, rb, ra)
                        for ref in (s1_ref, s2_ref):
                            va = ref[pl.ds(ra, _CH), :]
                            vb = ref[pl.ds(rb, _CH), :]
                            mn = jnp.minimum(va, vb)
                            mx = jnp.maximum(va, vb)
                            ref[pl.ds(amn, _CH), :] = mn
                            ref[pl.ds(amx, _CH), :] = mx
                    return 0

                jax.lax.fori_loop(0, n_pad // (2 * _CH), body, 0)

            local_a(s1_ref)
            local_a(s2_ref)

            if special:
                def level_body(t, _):
                    kk = jnp.int32(2 * _ROWB) << t

                    def pass_body(s, _):
                        jj = (kk // 2) >> s
                        global_pass(kk, jj,
                                    jnp.where(kk <= 1024, hw, 8192))
                        return 0

                    jax.lax.fori_loop(0, t + 1, pass_body, 0)
                    nb = jnp.where(kk <= 1024, data_blocks, data_blocks - 4)
                    local_b(s1_ref, kk, nb, False)
                    local_b(s2_ref, kk, nb, False)
                    return 0

                jax.lax.fori_loop(0, 5, level_body, 0)

                def shift_body(cc, _):
                    ra = 8192 - 1024 + cc * _CH
                    rb = 8192 + cc * _CH
                    for ref in (s1_ref, s2_ref):
                        va = ref[pl.ds(ra, _CH), :]
                        vb = -ref[pl.ds(rb, _CH), :]
                        ref[pl.ds(ra, _CH), :] = jnp.minimum(va, vb)
                        ref[pl.ds(rb, _CH), :] = jnp.maximum(va, vb)
                    return 0

                jax.lax.fori_loop(0, 1024 // _CH, shift_body, 0)

                def fin_pass(s, _):
                    global_pass(jnp.int32(n_pad), jnp.int32(4096) >> s,
                                jnp.int32(hw))
                    return 0

                jax.lax.fori_loop(0, 5, fin_pass, 0)

                def fin_local(ref):
                    def body(bi, _):
                        b0 = pl.multiple_of(bi * _ROWB, _ROWB)
                        vs = load_block(ref, b0)
                        vs = _merge_block(vs, n_proj, True)
                        store_block(ref, b0, vs)
                        return 0
                    jax.lax.fori_loop(0, data_blocks, body, 0)

                fin_local(s1_ref)

                n_i = nf.astype(jnp.int32)
                iota8 = jax.lax.broadcasted_iota(
                    jnp.int32, (_SUBL, n_proj), 0)

                def fin2_body(bi, acc):
                    b0 = pl.multiple_of(bi * _ROWB, _ROWB)
                    vs = load_block(s2_ref, b0)
                    vs = _merge_block(vs, n_proj, True)
                    for i, v in enumerate(vs):
                        row0 = b0 + i * _SUBL
                        x1 = s1_ref[pl.ds(row0, _SUBL), :]
                        valid = (iota8 + row0) < n_i
                        acc = acc + jnp.where(valid, jnp.abs(x1 - v), 0.0)
                    return acc

                accv = jax.lax.fori_loop(
                    0, data_blocks, fin2_body,
                    jnp.zeros((_SUBL, n_proj), jnp.float32))
                total = jnp.sum(accv)
            elif n_pad > _ROWB:
                n_levels = (n_pad // _ROWB).bit_length() - 1

                def level_body(t, _):
                    kk = jnp.int32(2 * _ROWB) << t

                    def pass_body(s, _):
                        jj = (kk // 2) >> s
                        global_pass(kk, jj, jnp.int32(n_pad))
                        return 0

                    jax.lax.fori_loop(0, t + 1, pass_body, 0)
                    local_b(s1_ref, kk, n_blocks, False)
                    local_b(s2_ref, kk, n_blocks, False)
                    return 0

                jax.lax.fori_loop(0, n_levels, level_body, 0)

            if not special:
                n_i = nf.astype(jnp.int32)
                riota = jax.lax.broadcasted_iota(
                    jnp.int32, (_CH, n_proj), 0)

                def red_body(t, acc):
                    r0 = t * _CH
                    x1 = s1_ref[pl.ds(r0, _CH), :]
                    x2 = s2_ref[pl.ds(r0, _CH), :]
                    valid = (riota + r0) < n_i
                    d = jnp.where(valid, jnp.abs(x1 - x2), 0.0)
                    return acc + jnp.sum(d)

                total = jax.lax.fori_loop(0, hw // _CH, red_body,
                                          jnp.float32(0.0))
            denom = jnp.maximum(nf * n_proj, 1.0)
            contrib = jnp.where(nf > 0, total / denom, 0.0)
            out_ref[0] = jnp.full((8, 128), contrib, jnp.float32)

    return kern


def kernel(content_features1, content_features2, gt_change_map, projections):
    b, c, h, w = content_features1.shape
    hw = h * w
    n_proj = projections.shape[1]
    ho, wo = gt_change_map.shape[2], gt_change_map.shape[3]

    r = (jnp.arange(h) * ho) // h
    cl = (jnp.arange(w) * wo) // w
    gt_small = gt_change_map[:, 0][:, r[:, None], cl[None, :]]
    maskf = (gt_small == 0).astype(jnp.float32).reshape(b, 1, hw)

    p1 = content_features1.reshape(b, c, hw)
    p2 = content_features2.reshape(b, c, hw)

    n_pad = 1 << (hw - 1).bit_length()
    n_pad = max(n_pad, _ROWB)
    c_chunks = 2 if c % 2 == 0 else (3 if c % 3 == 0 else 1)
    c_chunk = c // c_chunks

    kern = _make_kernel(hw, n_pad, c_chunk, c_chunks, n_proj)

    out = pl.pallas_call(
        kern,
        grid=(b, c_chunks),
        in_specs=[
            pl.BlockSpec((1, 1, hw), lambda i, ci: (i, 0, 0)),
            pl.BlockSpec((1, c_chunk, hw), lambda i, ci: (i, ci, 0)),
            pl.BlockSpec((1, c_chunk, hw), lambda i, ci: (i, ci, 0)),
            pl.BlockSpec((c, n_proj), lambda i, ci: (0, 0)),
        ],
        out_specs=pl.BlockSpec((1, 8, 128), lambda i, ci: (i, 0, 0)),
        out_shape=jax.ShapeDtypeStruct((b, 8, 128), jnp.float32),
        scratch_shapes=[
            pltpu.VMEM((n_pad, n_proj), jnp.float32),
            pltpu.VMEM((n_pad, n_proj), jnp.float32),
        ],
        compiler_params=pltpu.CompilerParams(
            dimension_semantics=("parallel", "arbitrary"),
        ),
    )(maskf, p1, p2, projections)

    return jnp.sum(out[:, 0, 0]) / b


# speedup vs baseline: 1.2459x; 1.0016x over previous
import jax
import jax.numpy as jnp
import numpy as np
from jax.experimental import pallas as pl
from jax.experimental.pallas import tpu as pltpu

_SUBL = 8
_ROWB = 256
_CH = 64
_BIG = 1.0e30


def _const_mask(fn, n_proj):
    rows = jax.lax.broadcasted_iota(jnp.int32, (_SUBL, n_proj), 0)
    return fn(rows)


def _subvreg_step(vs, k, j, n_proj, asc_static=None):
    lowm = _const_mask(lambda r: (r & j) == 0, n_proj)
    out = []
    for idx, v in enumerate(vs):
        if j == _SUBL // 2:
            p = pltpu.roll(v, j, 0)
        else:
            pm = pltpu.roll(v, _SUBL - j, 0)
            pp = pltpu.roll(v, j, 0)
            p = jnp.where(lowm, pm, pp)
        mn = jnp.minimum(v, p)
        mx = jnp.maximum(v, p)
        if asc_static is None:
            tm = _const_mask(
                lambda r: (((idx * _SUBL + r) & k) == 0) == ((r & j) == 0),
                n_proj)
        elif asc_static:
            tm = lowm
        else:
            tm = jnp.logical_not(lowm)
        out.append(jnp.where(tm, mn, mx))
    return out


def _pair_step(vs, k, j, asc_static=None):
    j8 = j // _SUBL
    out = list(vs)
    for a in range(len(vs)):
        if a & j8:
            continue
        b = a + j8
        mn = jnp.minimum(vs[a], vs[b])
        mx = jnp.maximum(vs[a], vs[b])
        if asc_static is None:
            asc = ((a * _SUBL) & k) == 0
        else:
            asc = asc_static
        if asc:
            out[a], out[b] = mn, mx
        else:
            out[a], out[b] = mx, mn
    return out


def _merge_block(vs, n_proj, asc):
    j = _ROWB // 2
    while j >= 1:
        if j < _SUBL:
            vs = _subvreg_step(vs, 0, j, n_proj, asc_static=asc)
        else:
            vs = _pair_step(vs, 0, j, asc_static=asc)
        j //= 2
    return vs


def _make_kernel(hw, n_pad, c_chunk, c_chunks, n_proj):
    nv = _ROWB // _SUBL
    hw_chunks = hw // 1024 if hw % 1024 == 0 else 1
    hw_chunk = hw // hw_chunks
    dn = (((0,), (0,)), ((), ()))
    special = (hw == 9216 and n_pad == 16384 and _ROWB == 256)

    def load_block(ref, b0):
        return [ref[pl.ds(b0 + i * _SUBL, _SUBL), :] for i in range(nv)]

    def store_block(ref, b0, vs):
        for i, v in enumerate(vs):
            ref[pl.ds(b0 + i * _SUBL, _SUBL), :] = v

    def kern(mask_ref, p1_ref, p2_ref, proj_ref, out_ref, s1_ref, s2_ref):
        ci = pl.program_id(1)

        pfull = proj_ref[...]
        pscale = 1.0 / jnp.maximum(
            jnp.sqrt(jnp.sum(pfull * pfull, axis=0, keepdims=True)), 1e-12)
        ones_row = jnp.ones((1, n_proj), jnp.float32)

        def mm_body(t, _):
            r0 = t * hw_chunk
            last_c = ci == c_chunks - 1
            mrow = mask_ref[0, :, pl.ds(r0, hw_chunk)]
            bslice = jax.lax.dot_general(
                jnp.where(last_c, (1.0 - mrow) * _BIG, 0.0), ones_row, dn,
                preferred_element_type=jnp.float32)
            if special:
                sgn = jnp.where(last_c & (t == hw_chunks - 1), -1.0, 1.0)
            else:
                sgn = 1.0
            for src, dst in ((p1_ref, s1_ref), (p2_ref, s2_ref)):
                lhs = src[0, :, pl.ds(r0, hw_chunk)]
                pc = proj_ref[pl.ds(ci * c_chunk, c_chunk), :] * pscale
                v = jax.lax.dot_general(
                    lhs, pc, dn, preferred_element_type=jnp.float32)

                @pl.when(ci == 0)
                def _():
                    dst[pl.ds(r0, hw_chunk), :] = (v + bslice) * sgn

                @pl.when(ci > 0)
                def _():
                    dst[pl.ds(r0, hw_chunk), :] = (
                        dst[pl.ds(r0, hw_chunk), :] + v + bslice) * sgn
            return 0

        jax.lax.fori_loop(0, hw_chunks, mm_body, 0)

        @pl.when(ci == 0)
        def _init_pad():
            if n_pad > hw and not special:
                pad = jnp.full((n_pad - hw, n_proj), jnp.inf, jnp.float32)
                s1_ref[hw:, :] = pad
                s2_ref[hw:, :] = pad

        @pl.when(ci == c_chunks - 1)
        def _finish():
            maskf = mask_ref[0]
            nf = jnp.sum(maskf)

            n_blocks = n_pad // _ROWB
            data_blocks = -(-hw // _ROWB)

            def local_a(ref):
                def body(bi, _):
                    b0 = pl.multiple_of(bi * _ROWB, _ROWB)
                    vs = load_block(ref, b0)
                    k = 2
                    while k <= _ROWB // 2:
                        j = k // 2
                        while j >= 1:
                            if j < _SUBL:
                                vs = _subvreg_step(vs, k, j, n_proj)
                            else:
                                vs = _pair_step(vs, k, j)
                            j //= 2
                        k *= 2
                    asc = (b0 & _ROWB) == 0
                    vs = jax.lax.cond(
                        asc,
                        lambda *a: tuple(_merge_block(list(a), n_proj, True)),
                        lambda *a: tuple(_merge_block(list(a), n_proj, False)),
                        *vs)
                    store_block(ref, b0, list(vs))
                    return 0
                nb = data_blocks if special else n_blocks
                jax.lax.fori_loop(0, nb, body, 0)

            def local_b(ref, kk, nb, remap):
                def body(bi, _):
                    if remap:
                        bi = jnp.where(
                            (kk == 8192) & (bi >= data_blocks),
                            bi + (n_blocks - data_blocks - 4), bi)
                    b0 = pl.multiple_of(bi * _ROWB, _ROWB)
                    asc = (b0 & kk) == 0
                    vs = load_block(ref, b0)
                    vs = jax.lax.cond(
                        asc,
                        lambda *a: tuple(_merge_block(list(a), n_proj, True)),
                        lambda *a: tuple(_merge_block(list(a), n_proj, False)),
                        *vs)
                    store_block(ref, b0, list(vs))
                    return 0
                jax.lax.fori_loop(0, nb, body, 0)

            def global_pass(kk, jj, limit):
                jc = jj // _CH

                def body(t, _):
                    lo = jnp.bitwise_and(t, jc - 1)
                    ra = ((t - lo) * 2 + lo) * _CH
                    rb = ra + jj

                    @pl.when(rb < limit)
                    def _():
                        asc = (ra & kk) == 0
                        amn = jnp.where(asc, ra, rb)
                        amx = jnp.where(asc, rb, ra)
                        for ref in (s1_ref, s2_ref):
                            va = ref[pl.ds(ra, _CH), :]
                            vb = ref[pl.ds(rb, _CH), :]
                            mn = jnp.minimum(va, vb)
                            mx = jnp.maximum(va, vb)
                            ref[pl.ds(amn, _CH), :] = mn
                            ref[pl.ds(amx, _CH), :] = mx
                    return 0

                jax.lax.fori_loop(0, n_pad // (2 * _CH), body, 0)

            local_a(s1_ref)
            local_a(s2_ref)

            if special:
                def level_body(t, _):
                    kk = jnp.int32(2 * _ROWB) << t

                    def pass_body(s, _):
                        jj = (kk // 2) >> s
                        global_pass(kk, jj,
                                    jnp.where(kk <= 1024, hw, 8192))
                        return 0

                    jax.lax.fori_loop(0, t + 1, pass_body, 0)
                    nb = jnp.where(kk <= 1024, data_blocks, data_blocks - 4)
                    local_b(s1_ref, kk, nb, False)
                    local_b(s2_ref, kk, nb, False)
                    return 0

                jax.lax.fori_loop(0, 5, level_body, 0)

                def shift_body(cc, _):
                    ra = 8192 - 1024 + cc * _CH
                    rb = 8192 + cc * _CH
                    for ref in (s1_ref, s2_ref):
                        va = ref[pl.ds(ra, _CH), :]
                        vb = -ref[pl.ds(rb, _CH), :]
                        ref[pl.ds(ra, _CH), :] = jnp.minimum(va, vb)
                        ref[pl.ds(rb, _CH), :] = jnp.maximum(va, vb)
                    return 0

                jax.lax.fori_loop(0, 1024 // _CH, shift_body, 0)

                def fin_pass(s, _):
                    global_pass(jnp.int32(n_pad), jnp.int32(4096) >> s,
                                jnp.int32(hw))
                    return 0

                jax.lax.fori_loop(0, 5, fin_pass, 0)

                def fin_local(ref):
                    def body(bi, _):
                        b0 = pl.multiple_of(bi * _ROWB, _ROWB)
                        vs = load_block(ref, b0)
                        vs = _merge_block(vs, n_proj, True)
                        store_block(ref, b0, vs)
                        return 0
                    jax.lax.fori_loop(0, data_blocks, body, 0)

                fin_local(s1_ref)

                n_i = nf.astype(jnp.int32)
                iota8 = jax.lax.broadcasted_iota(
                    jnp.int32, (_SUBL, n_proj), 0)

                def fin2_body(bi, acc):
                    b0 = pl.multiple_of(bi * _ROWB, _ROWB)
                    vs = load_block(s2_ref, b0)
                    vs = _merge_block(vs, n_proj, True)
                    for i, v in enumerate(vs):
                        row0 = b0 + i * _SUBL
                        x1 = s1_ref[pl.ds(row0, _SUBL), :]
                        valid = (iota8 + row0) < n_i
                        acc = acc + jnp.where(valid, jnp.abs(x1 - v), 0.0)
                    return acc

                accv = jax.lax.fori_loop(
                    0, data_blocks, fin2_body,
                    jnp.zeros((_SUBL, n_proj), jnp.float32))
                total = jnp.sum(accv)
            elif n_pad > _ROWB:
                n_levels = (n_pad // _ROWB).bit_length() - 1

                def level_body(t, _):
                    kk = jnp.int32(2 * _ROWB) << t

                    def pass_body(s, _):
                        jj = (kk // 2) >> s
                        global_pass(kk, jj, jnp.int32(n_pad))
                        return 0

                    jax.lax.fori_loop(0, t + 1, pass_body, 0)
                    local_b(s1_ref, kk, n_blocks, False)
                    local_b(s2_ref, kk, n_blocks, False)
                    return 0

                jax.lax.fori_loop(0, n_levels, level_body, 0)

            if not special:
                n_i = nf.astype(jnp.int32)
                riota = jax.lax.broadcasted_iota(
                    jnp.int32, (_CH, n_proj), 0)

                def red_body(t, acc):
                    r0 = t * _CH
                    x1 = s1_ref[pl.ds(r0, _CH), :]
                    x2 = s2_ref[pl.ds(r0, _CH), :]
                    valid = (riota + r0) < n_i
                    d = jnp.where(valid, jnp.abs(x1 - x2), 0.0)
                    return acc + jnp.sum(d)

                total = jax.lax.fori_loop(0, hw // _CH, red_body,
                                          jnp.float32(0.0))
            denom = jnp.maximum(nf * n_proj, 1.0)
            contrib = jnp.where(nf > 0, total / denom, 0.0)
            out_ref[0] = jnp.full((8, 128), contrib, jnp.float32)

    return kern


def kernel(content_features1, content_features2, gt_change_map, projections):
    b, c, h, w = content_features1.shape
    hw = h * w
    n_proj = projections.shape[1]
    ho, wo = gt_change_map.shape[2], gt_change_map.shape[3]

    r = (jnp.arange(h) * ho) // h
    cl = (jnp.arange(w) * wo) // w
    gt_small = gt_change_map[:, 0][:, r[:, None], cl[None, :]]
    maskf = (gt_small == 0).astype(jnp.float32).reshape(b, 1, hw)

    p1 = content_features1.reshape(b, c, hw)
    p2 = content_features2.reshape(b, c, hw)

    n_pad = 1 << (hw - 1).bit_length()
    n_pad = max(n_pad, _ROWB)
    c_chunks = 2 if c % 2 == 0 else (3 if c % 3 == 0 else 1)
    c_chunk = c // c_chunks

    kern = _make_kernel(hw, n_pad, c_chunk, c_chunks, n_proj)

    out = pl.pallas_call(
        kern,
        grid=(b, c_chunks),
        in_specs=[
            pl.BlockSpec((1, 1, hw), lambda i, ci: (i, 0, 0)),
            pl.BlockSpec((1, c_chunk, hw), lambda i, ci: (i, ci, 0)),
            pl.BlockSpec((1, c_chunk, hw), lambda i, ci: (i, ci, 0)),
            pl.BlockSpec((c, n_proj), lambda i, ci: (0, 0)),
        ],
        out_specs=pl.BlockSpec((1, 8, 128), lambda i, ci: (i, 0, 0)),
        out_shape=jax.ShapeDtypeStruct((b, 8, 128), jnp.float32),
        scratch_shapes=[
            pltpu.VMEM((n_pad, n_proj), jnp.float32),
            pltpu.VMEM((n_pad, n_proj), jnp.float32),
        ],
        compiler_params=pltpu.CompilerParams(
            dimension_semantics=("parallel", "arbitrary"),
        ),
    )(maskf, p1, p2, projections)

    return jnp.sum(out[:, 0, 0]) / b
